# Initial kernel scaffold; baseline (speedup 1.0000x reference)
import jax
import jax.numpy as jnp
from jax.experimental import pallas as pl


def kernel(x_genes, edge_index_genes, edge_index_patients, clinical_features, closeness_scores, eigenvector_scores, betweenness_scores, params):
    raise NotImplementedError("write your pallas kernel here")


# speedup vs baseline: 244.9021x; 244.9021x over previous
import jax
import jax.numpy as jnp
from jax.experimental import pallas as pl
from jax.experimental.pallas import tpu as pltpu

B = 128
NG = 10000
NGP = 10240
NC = 32
HID = 512
EP = 2048

_CH = 1024
_HIGH = jax.lax.Precision.HIGHEST


def _dot(a, b):
    return jax.lax.dot_general(a, b, (((1,), (0,)), ((), ())),
                               preferred_element_type=jnp.float32,
                               precision=_HIGH)


def _xp_kernel(scal_ref, xg_ref, clo_ref, eig_ref, bet_ref, wt_ref, bias_ref,
               out_ref):
    j = pl.program_id(0)
    cs = (scal_ref[0] * clo_ref[...] + scal_ref[1] * eig_ref[...]
          + scal_ref[2] * bet_ref[...])
    acc = _dot(xg_ref[...] * cs, wt_ref[...])

    @pl.when(j == 0)
    def _():
        out_ref[...] = acc + bias_ref[...]

    @pl.when(j != 0)
    def _():
        out_ref[...] += acc


def _ln(x, g, b):
    m = jnp.mean(x, axis=-1, keepdims=True)
    v = jnp.mean((x - m) ** 2, axis=-1, keepdims=True)
    return (x - m) / jnp.sqrt(v + 1e-5) * g + b


def _leaky(x, s):
    return jnp.where(x >= 0, x, s * x)


def _gat_kernel(concat):
    def body(x_ref, src_ref, dst_ref, dstl_ref, wlT, bl, wrT, br, att, bias,
             out_ref):
        iota_l = jax.lax.broadcasted_iota(jnp.int32, (EP, B), 1)
        iota_s = jax.lax.broadcasted_iota(jnp.int32, (B, EP), 0)
        mask_dst = dst_ref[...] == iota_l
        p_src = (src_ref[...] == iota_l).astype(jnp.float32)
        p_dst = mask_dst.astype(jnp.float32)
        p_dstT = (dstl_ref[...] == iota_s).astype(jnp.float32)

        xl = _dot(x_ref[...], wlT[...]) + bl[...]
        xr = _dot(x_ref[...], wrT[...]) + br[...]
        outs = []
        for h in range(2):
            sl = slice(h * HID, (h + 1) * HID)
            e_src = _dot(p_src, xl[:, sl])
            e_dst = _dot(p_dst, xr[:, sl])
            eh = _leaky(e_src + e_dst, 0.2)
            logit = jnp.sum(eh * att[:, sl], axis=1, keepdims=True)
            masked = jnp.where(mask_dst, logit, -1e30)
            m = jnp.max(masked, axis=0, keepdims=True)
            m = jnp.where(m > -1e29, m, 0.0)
            amax_e = jnp.sum(p_dst * m, axis=1, keepdims=True)
            ex = jnp.exp(logit - amax_e)
            den = _dot(p_dstT, ex)
            den_e = _dot(p_dst, den)
            a = ex / (den_e + 1e-16)
            outs.append(_dot(p_dstT, a * e_src))
        if concat:
            out = jnp.concatenate(outs, axis=1)
        else:
            out = (outs[0] + outs[1]) * 0.5
        out_ref[...] = out + bias[...]

    return body


def _tail_kernel(h_ref, clin_ref, fpT, fpb, wvT, wvb, woT, wob, wcT, wcb,
                 f1aT, f1bT, f1cT, f1b, lnfg, lnfb,
                 f2T, f2b, lnf2g, lnf2b, f3T, f3b,
                 r1aT, r1bT, r1cT, r1b, r2T, r2b, r3T, r3b,
                 out1_ref, out2_ref):
    x_pat = _dot(h_ref[...], fpT[...]) + fpb[...]
    v = _dot(x_pat, wvT[...]) + wvb[...]
    cross = _dot(v, woT[...]) + wob[...]
    clin_e = _dot(clin_ref[...], wcT[...]) + wcb[...]

    z = (_dot(cross, f1aT[...]) + _dot(x_pat, f1bT[...])
         + _dot(clin_e, f1cT[...]) + f1b[...])
    z = jnp.maximum(_ln(z, lnfg[...], lnfb[...]), 0.0)
    z = _dot(z, f2T[...]) + f2b[...]
    z = jnp.maximum(_ln(z, lnf2g[...], lnf2b[...]), 0.0)
    out1_ref[...] = _dot(z, f3T[...]) + f3b[...]

    r = (_dot(cross, r1aT[...]) + _dot(x_pat, r1bT[...])
         + _dot(clin_e, r1cT[...]) + r1b[...])
    r = jnp.maximum(r, 0.0)
    r = jnp.maximum(_dot(r, r2T[...]) + r2b[...], 0.0)
    out2_ref[...] = _dot(r, r3T[...]) + r3b[...]


def _ln_leaky_kernel(x_ref, g_ref, b_ref, out_ref):
    out_ref[...] = _leaky(_ln(x_ref[...], g_ref[...], b_ref[...]), 0.01)


def kernel(x_genes, edge_index_genes, edge_index_patients, clinical_features,
           closeness_scores, eigenvector_scores, betweenness_scores, params):
    P = params
    f32 = jnp.float32
    scal = jnp.stack([P['alpha'], P['beta'], P['gamma']]).astype(f32)
    pad = ((0, 0), (0, NGP - NG))
    xg_p = jnp.pad(x_genes, pad)
    clo_p = jnp.pad(closeness_scores, pad)
    eig_p = jnp.pad(eigenvector_scores, pad)
    bet_p = jnp.pad(betweenness_scores, pad)
    wt = jnp.pad(P['fc_g2p']['w'].T, ((0, NGP - NG), (0, 0)))
    bias = P['fc_g2p']['b'].reshape(1, HID)

    xp = pl.pallas_call(
        _xp_kernel,
        grid=(NGP // _CH,),
        in_specs=[
            pl.BlockSpec(memory_space=pltpu.SMEM),
            pl.BlockSpec((B, _CH), lambda j: (0, j)),
            pl.BlockSpec((B, _CH), lambda j: (0, j)),
            pl.BlockSpec((B, _CH), lambda j: (0, j)),
            pl.BlockSpec((B, _CH), lambda j: (0, j)),
            pl.BlockSpec((_CH, HID), lambda j: (j, 0)),
            pl.BlockSpec((1, HID), lambda j: (0, 0)),
        ],
        out_specs=pl.BlockSpec((B, HID), lambda j: (0, 0)),
        out_shape=jax.ShapeDtypeStruct((B, HID), f32),
    )(scal, xg_p, clo_p, eig_p, bet_p, wt, bias)

    src = edge_index_patients[0].reshape(EP, 1)
    dst = edge_index_patients[1].reshape(EP, 1)
    dstl = edge_index_patients[1].reshape(1, EP)

    def gat_layer(x, p, concat):
        out_d = 2 * HID if concat else HID
        return pl.pallas_call(
            _gat_kernel(concat),
            out_shape=jax.ShapeDtypeStruct((B, out_d), f32),
        )(x, src, dst, dstl, p['l']['w'].T, p['l']['b'].reshape(1, -1),
          p['r']['w'].T, p['r']['b'].reshape(1, -1),
          p['att'].reshape(1, -1), p['bias'].reshape(1, -1))

    def ln_leaky(x, p):
        return pl.pallas_call(
            _ln_leaky_kernel,
            out_shape=jax.ShapeDtypeStruct(x.shape, f32),
        )(x, p['g'].reshape(1, -1), p['b'].reshape(1, -1))

    h = gat_layer(xp, P['gp1'], True)
    h = ln_leaky(h, P['bn_p1'])
    h = gat_layer(h, P['gp2'], False)
    h = ln_leaky(h, P['bn_p2'])
    res = h
    h = gat_layer(h, P['gp3'], False)
    h = ln_leaky(h, P['bn_p3']) + res

    E = HID
    wv = P['mha']['in_w'][2 * E:]
    bv = P['mha']['in_b'][2 * E:].reshape(1, E)
    f1w = P['fc1']['w']
    r1w = P['risk1']['w']

    def lnw(p):
        return (p['g'].reshape(1, -1), p['b'].reshape(1, -1))

    out1, out2 = pl.pallas_call(
        _tail_kernel,
        out_shape=(jax.ShapeDtypeStruct((B, 1), f32),
                   jax.ShapeDtypeStruct((B, 1), f32)),
    )(h, clinical_features,
      P['fc_patient']['w'].T, P['fc_patient']['b'].reshape(1, -1),
      wv.T, bv, P['mha']['out']['w'].T, P['mha']['out']['b'].reshape(1, -1),
      P['clin']['w'].T, P['clin']['b'].reshape(1, -1),
      f1w[:, :E].T, f1w[:, E:2 * E].T, f1w[:, 2 * E:].T,
      P['fc1']['b'].reshape(1, -1), *lnw(P['bn_fc']),
      P['fc2']['w'].T, P['fc2']['b'].reshape(1, -1), *lnw(P['bn_fc2']),
      P['fc3']['w'].T, P['fc3']['b'].reshape(1, -1),
      r1w[:, :E].T, r1w[:, E:2 * E].T, r1w[:, 2 * E:].T,
      P['risk1']['b'].reshape(1, -1),
      P['risk2']['w'].T, P['risk2']['b'].reshape(1, -1),
      P['risk3']['w'].T, P['risk3']['b'].reshape(1, -1))

    return out1, out2[:, 0]


# speedup vs baseline: 246.7405x; 1.0075x over previous
import jax
import jax.numpy as jnp
from jax.experimental import pallas as pl
from jax.experimental.pallas import tpu as pltpu

B = 128
NG = 10000
NGP = 10240
NC = 32
HID = 512
EP = 2048

_CH = 1024
_HIGH = jax.lax.Precision.HIGHEST


def _dot(a, b):
    return jax.lax.dot_general(a, b, (((1,), (0,)), ((), ())),
                               preferred_element_type=jnp.float32,
                               precision=_HIGH)


def _xp_kernel(scal_ref, xg_ref, clo_ref, eig_ref, bet_ref, wt_ref, bias_ref,
               out_ref):
    j = pl.program_id(0)
    cs = (scal_ref[0] * clo_ref[...] + scal_ref[1] * eig_ref[...]
          + scal_ref[2] * bet_ref[...])
    acc = _dot(xg_ref[...] * cs, wt_ref[...])

    @pl.when(j == 0)
    def _():
        out_ref[...] = acc + bias_ref[...]

    @pl.when(j != 0)
    def _():
        out_ref[...] += acc


def _ln(x, g, b):
    m = jnp.mean(x, axis=-1, keepdims=True)
    v = jnp.mean((x - m) ** 2, axis=-1, keepdims=True)
    return (x - m) / jnp.sqrt(v + 1e-5) * g + b


def _leaky(x, s):
    return jnp.where(x >= 0, x, s * x)


def _gat_kernel(concat, residual):
    def body(x_ref, src_ref, dst_ref, dstl_ref, wlT, bl, wrT, br, att, bias,
             lng, lnb, out_ref):
        iota_l = jax.lax.broadcasted_iota(jnp.int32, (EP, B), 1)
        iota_s = jax.lax.broadcasted_iota(jnp.int32, (B, EP), 0)
        mask_dst = dst_ref[...] == iota_l
        p_src = (src_ref[...] == iota_l).astype(jnp.float32)
        p_dst = mask_dst.astype(jnp.float32)
        p_dstT = (dstl_ref[...] == iota_s).astype(jnp.float32)

        xl = _dot(x_ref[...], wlT[...]) + bl[...]
        xr = _dot(x_ref[...], wrT[...]) + br[...]
        outs = []
        for h in range(2):
            sl = slice(h * HID, (h + 1) * HID)
            e_src = _dot(p_src, xl[:, sl])
            e_dst = _dot(p_dst, xr[:, sl])
            eh = _leaky(e_src + e_dst, 0.2)
            logit = jnp.sum(eh * att[:, sl], axis=1, keepdims=True)
            masked = jnp.where(mask_dst, logit, -1e30)
            m = jnp.max(masked, axis=0, keepdims=True)
            m = jnp.where(m > -1e29, m, 0.0)
            amax_e = jnp.sum(p_dst * m, axis=1, keepdims=True)
            ex = jnp.exp(logit - amax_e)
            den = _dot(p_dstT, ex)
            den_e = _dot(p_dst, den)
            a = ex / (den_e + 1e-16)
            outs.append(_dot(p_dstT, a * e_src))
        if concat:
            out = jnp.concatenate(outs, axis=1)
        else:
            out = (outs[0] + outs[1]) * 0.5
        out = _leaky(_ln(out + bias[...], lng[...], lnb[...]), 0.01)
        if residual:
            out = out + x_ref[...]
        out_ref[...] = out

    return body


def _tail_kernel(h_ref, clin_ref, fpT, fpb, wvT, wvb, woT, wob, wcT, wcb,
                 f1aT, f1bT, f1cT, f1b, lnfg, lnfb,
                 f2T, f2b, lnf2g, lnf2b, f3T, f3b,
                 r1aT, r1bT, r1cT, r1b, r2T, r2b, r3T, r3b,
                 out1_ref, out2_ref):
    x_pat = _dot(h_ref[...], fpT[...]) + fpb[...]
    v = _dot(x_pat, wvT[...]) + wvb[...]
    cross = _dot(v, woT[...]) + wob[...]
    clin_e = _dot(clin_ref[...], wcT[...]) + wcb[...]

    z = (_dot(cross, f1aT[...]) + _dot(x_pat, f1bT[...])
         + _dot(clin_e, f1cT[...]) + f1b[...])
    z = jnp.maximum(_ln(z, lnfg[...], lnfb[...]), 0.0)
    z = _dot(z, f2T[...]) + f2b[...]
    z = jnp.maximum(_ln(z, lnf2g[...], lnf2b[...]), 0.0)
    out1_ref[...] = _dot(z, f3T[...]) + f3b[...]

    r = (_dot(cross, r1aT[...]) + _dot(x_pat, r1bT[...])
         + _dot(clin_e, r1cT[...]) + r1b[...])
    r = jnp.maximum(r, 0.0)
    r = jnp.maximum(_dot(r, r2T[...]) + r2b[...], 0.0)
    out2_ref[...] = _dot(r, r3T[...]) + r3b[...]


def kernel(x_genes, edge_index_genes, edge_index_patients, clinical_features,
           closeness_scores, eigenvector_scores, betweenness_scores, params):
    P = params
    f32 = jnp.float32
    scal = jnp.stack([P['alpha'], P['beta'], P['gamma']]).astype(f32)
    pad = ((0, 0), (0, NGP - NG))
    xg_p = jnp.pad(x_genes, pad)
    clo_p = jnp.pad(closeness_scores, pad)
    eig_p = jnp.pad(eigenvector_scores, pad)
    bet_p = jnp.pad(betweenness_scores, pad)
    wt = jnp.pad(P['fc_g2p']['w'].T, ((0, NGP - NG), (0, 0)))
    bias = P['fc_g2p']['b'].reshape(1, HID)

    xp = pl.pallas_call(
        _xp_kernel,
        grid=(NGP // _CH,),
        in_specs=[
            pl.BlockSpec(memory_space=pltpu.SMEM),
            pl.BlockSpec((B, _CH), lambda j: (0, j)),
            pl.BlockSpec((B, _CH), lambda j: (0, j)),
            pl.BlockSpec((B, _CH), lambda j: (0, j)),
            pl.BlockSpec((B, _CH), lambda j: (0, j)),
            pl.BlockSpec((_CH, HID), lambda j: (j, 0)),
            pl.BlockSpec((1, HID), lambda j: (0, 0)),
        ],
        out_specs=pl.BlockSpec((B, HID), lambda j: (0, 0)),
        out_shape=jax.ShapeDtypeStruct((B, HID), f32),
    )(scal, xg_p, clo_p, eig_p, bet_p, wt, bias)

    src = edge_index_patients[0].reshape(EP, 1)
    dst = edge_index_patients[1].reshape(EP, 1)
    dstl = edge_index_patients[1].reshape(1, EP)

    def gat_layer(x, p, lnp, concat, residual=False):
        out_d = 2 * HID if concat else HID
        return pl.pallas_call(
            _gat_kernel(concat, residual),
            out_shape=jax.ShapeDtypeStruct((B, out_d), f32),
        )(x, src, dst, dstl, p['l']['w'].T, p['l']['b'].reshape(1, -1),
          p['r']['w'].T, p['r']['b'].reshape(1, -1),
          p['att'].reshape(1, -1), p['bias'].reshape(1, -1),
          lnp['g'].reshape(1, -1), lnp['b'].reshape(1, -1))

    h = gat_layer(xp, P['gp1'], P['bn_p1'], True)
    h = gat_layer(h, P['gp2'], P['bn_p2'], False)
    h = gat_layer(h, P['gp3'], P['bn_p3'], False, residual=True)

    E = HID
    wv = P['mha']['in_w'][2 * E:]
    bv = P['mha']['in_b'][2 * E:].reshape(1, E)
    f1w = P['fc1']['w']
    r1w = P['risk1']['w']

    def lnw(p):
        return (p['g'].reshape(1, -1), p['b'].reshape(1, -1))

    out1, out2 = pl.pallas_call(
        _tail_kernel,
        out_shape=(jax.ShapeDtypeStruct((B, 1), f32),
                   jax.ShapeDtypeStruct((B, 1), f32)),
    )(h, clinical_features,
      P['fc_patient']['w'].T, P['fc_patient']['b'].reshape(1, -1),
      wv.T, bv, P['mha']['out']['w'].T, P['mha']['out']['b'].reshape(1, -1),
      P['clin']['w'].T, P['clin']['b'].reshape(1, -1),
      f1w[:, :E].T, f1w[:, E:2 * E].T, f1w[:, 2 * E:].T,
      P['fc1']['b'].reshape(1, -1), *lnw(P['bn_fc']),
      P['fc2']['w'].T, P['fc2']['b'].reshape(1, -1), *lnw(P['bn_fc2']),
      P['fc3']['w'].T, P['fc3']['b'].reshape(1, -1),
      r1w[:, :E].T, r1w[:, E:2 * E].T, r1w[:, 2 * E:].T,
      P['risk1']['b'].reshape(1, -1),
      P['risk2']['w'].T, P['risk2']['b'].reshape(1, -1),
      P['risk3']['w'].T, P['risk3']['b'].reshape(1, -1))

    return out1, out2[:, 0]
